# baseline (speedup 1.0000x reference)
import jax
import jax.numpy as jnp
from jax import lax
from jax.experimental import pallas as pl
from jax.experimental.pallas import tpu as pltpu
from jax.experimental.pallas import tpu_sc as plsc

B, H, W, C = 8, 112, 112, 96
IMG_IN = H * W * C
IMG_OUT = 4 * IMG_IN
TOTAL_OUT = B * IMG_OUT

NC, NS = 2, 16
NREG = 3
REGION = IMG_OUT // NREG
TASKS = (B // NC) * NREG

WCH = 16
CHUNK = WCH * C
PAIRS_PER_TILE = IMG_IN // NS
NCHUNK = PAIRS_PER_TILE // CHUNK
CH_PER_BATCH = IMG_IN // CHUNK
WSPLIT = W // WCH

TRASH_PER_TILE = 2048
TRASH = NS * TRASH_PER_TILE
SPMEM_WORDS = REGION + TRASH

OUT_PER_TILE = REGION // NS
CPBUF = 6272
NCP = OUT_PER_TILE // CPBUF


def _body(val_hbm, idx_hbm, out_hbm, shared,
          idxb0, idxb1, valb0, valb1, sidx0, sidx1, sval0, sval1,
          cbuf, sl0, sl1, ss0, ss1, zsem):
    c = lax.axis_index("c")
    t = lax.axis_index("s")
    lane = lax.iota(jnp.int32, 16)
    idxb = (idxb0, idxb1)
    valb = (valb0, valb1)
    sidx = (sidx0, sidx1)
    sval = (sval0, sval1)
    sl = (sl0, sl1)
    ss = (ss0, ss1)

    def zfill(i, carry):
        cbuf[pl.ds(i * 16, 16)] = jnp.zeros((16,), jnp.float32)
        return carry

    lax.fori_loop(0, CPBUF // 16, zfill, 0)

    def task_body(r, carry):
        b = c * (B // NC) + r // NREG
        q = r % NREG
        lo = q * REGION
        hi = lo + REGION

        def start_load(k):
            p = k % 2
            g = k * NS + t
            h2 = g // WSPLIT
            w0 = (g % WSPLIT) * WCH
            h1_ = pltpu.async_copy(
                idx_hbm.at[b, h2, pl.ds(w0, WCH), :], idxb[p], sl[p])
            h2_ = pltpu.async_copy(
                val_hbm.at[b, h2, pl.ds(w0, WCH), :], valb[p], sl[p])
            return h1_, h2_

        h_load = start_load(0)
        h_zero = []
        for k in range(NCP):
            h_zero.append(pltpu.async_copy(
                cbuf, shared.at[pl.ds(t * OUT_PER_TILE + k * CPBUF, CPBUF)],
                zsem))
        for h in h_zero:
            h.wait()
        plsc.subcore_barrier()

        h_scat = None
        for k in range(NCHUNK):
            p = k % 2
            h_load[0].wait()
            h_load[1].wait()

            def vec_body(j, carry3, _ib=idxb[p], _vb=valb[p],
                         _si=sidx[p], _sv=sval[p]):
                for v in range(C // 16):
                    iv = _ib[j, pl.ds(v * 16, 16)]
                    fv = _vb[j, pl.ds(v * 16, 16)]
                    m = (iv >= lo) & (iv < hi)
                    pos = j * C + v * 16
                    tr = (REGION + t * TRASH_PER_TILE
                          + (pos & (TRASH_PER_TILE - 16)) + lane)
                    _si[pl.ds(pos, 16)] = jnp.where(m, iv - lo, tr)
                    _sv[pl.ds(pos, 16)] = fv
                return carry3

            lax.fori_loop(0, WCH, vec_body, 0)
            if h_scat is not None:
                h_scat.wait()
            h_scat = pltpu.async_copy(sval[p], shared.at[sidx[p]], ss[p],
                                      add=True)
            if k + 1 < NCHUNK:
                h_load = start_load(k + 1)
        h_scat.wait()
        plsc.subcore_barrier()

        g0 = b * IMG_OUT + lo + t * OUT_PER_TILE
        pltpu.sync_copy(shared.at[pl.ds(t * OUT_PER_TILE, OUT_PER_TILE)],
                        out_hbm.at[pl.ds(g0, OUT_PER_TILE)])
        return carry

    lax.fori_loop(0, TASKS, task_body, 0)


def kernel(max_values, argmax):
    idx = argmax.astype(jnp.int32)
    run = pl.kernel(
        _body,
        out_type=jax.ShapeDtypeStruct((TOTAL_OUT,), jnp.float32),
        mesh=plsc.VectorSubcoreMesh(
            core_axis_name="c", subcore_axis_name="s",
            num_cores=NC, num_subcores=NS),
        scratch_types=[
            pltpu.MemorySpace.VMEM_SHARED((SPMEM_WORDS,), jnp.float32),
            pltpu.MemorySpace.VMEM((WCH, C), jnp.int32),
            pltpu.MemorySpace.VMEM((WCH, C), jnp.int32),
            pltpu.MemorySpace.VMEM((WCH, C), jnp.float32),
            pltpu.MemorySpace.VMEM((WCH, C), jnp.float32),
            pltpu.MemorySpace.VMEM((CHUNK,), jnp.int32),
            pltpu.MemorySpace.VMEM((CHUNK,), jnp.int32),
            pltpu.MemorySpace.VMEM((CHUNK,), jnp.float32),
            pltpu.MemorySpace.VMEM((CHUNK,), jnp.float32),
            pltpu.MemorySpace.VMEM((CPBUF,), jnp.float32),
            pltpu.SemaphoreType.DMA,
            pltpu.SemaphoreType.DMA,
            pltpu.SemaphoreType.DMA,
            pltpu.SemaphoreType.DMA,
            pltpu.SemaphoreType.DMA,
        ],
    )
    out = run(max_values, idx)
    return out.reshape(B, 2 * H, 2 * W, C)
